# baseline (speedup 1.0000x reference)
import functools

import jax
import jax.numpy as jnp
from jax import lax
from jax.experimental import pallas as pl
from jax.experimental.pallas import tpu as pltpu
from jax.experimental.pallas import tpu_sc as plsc

B = 16384
NF = 26
V = 100000
E = 32
D_DENSE = 13
H = 1028

NW = 32
RB = B // 8
RBW = RB // NW
KRB = 4
MSTEPS = RBW // KRB
RPB = 224
IRW = RBW * RPB // 128
CPS = KRB * RPB // 128


def _sc_gather_body(tbl_hbm, idx_hbm, out_hbm, imv, g0, g1, gsem, os0, os1):
    wid = lax.axis_index("s") * 2 + lax.axis_index("c")
    pltpu.sync_copy(idx_hbm.at[pl.ds(wid * IRW, IRW)], imv)

    slots = (g0, g1)
    osems = (os0, os1)
    pending = [None, None]
    for i in range(MSTEPS):
        slot = i % 2
        if pending[slot] is not None:
            pending[slot].wait()
        g = slots[slot]
        descs = []
        for j in range(CPS):
            descs.append(pltpu.async_copy(
                tbl_hbm.at[imv.at[i * CPS + j]],
                g.at[pl.ds(j * 128, 128)], gsem))
        for d in descs:
            d.wait()
        out_row = (wid * RBW + i * KRB) * RPB
        pending[slot] = pltpu.async_copy(
            g, out_hbm.at[pl.ds(out_row, KRB * RPB)], osems[slot])
    for p in pending:
        if p is not None:
            p.wait()


@functools.partial(
    pl.kernel,
    out_type=jax.ShapeDtypeStruct((RB * RPB, E), jnp.float32),
    mesh=plsc.VectorSubcoreMesh(core_axis_name="c", subcore_axis_name="s"),
    compiler_params=pltpu.CompilerParams(use_tc_tiling_on_sc=False),
    scratch_types=[
        pltpu.VMEM((IRW, 128), jnp.int32),
        pltpu.VMEM((KRB * RPB, E), jnp.float32),
        pltpu.VMEM((KRB * RPB, E), jnp.float32),
        pltpu.SemaphoreType.DMA,
        pltpu.SemaphoreType.DMA,
        pltpu.SemaphoreType.DMA,
    ],
)
def _sc_gather(tbl_hbm, idx_hbm, out_hbm, imv, g0, g1, gsem, os0, os1):
    _sc_gather_body(tbl_hbm, idx_hbm, out_hbm, imv, g0, g1, gsem, os0, os1)


def _mlp_body(x_ref, nc_ref, w1_ref, w1d_ref, b1_ref, w2_ref, o_ref):
    bm8 = x_ref.shape[0]
    acc = jnp.zeros((bm8 * 8, H), jnp.float32)
    for c in range(7):
        xc = x_ref[:, c].reshape(bm8 * 8, 128)
        if c == 6:
            lane = lax.broadcasted_iota(jnp.int32, xc.shape, 1)
            xc = jnp.where(lane < 64, xc, 0.0)
        acc = acc + jnp.dot(xc.astype(jnp.bfloat16), w1_ref[c],
                            preferred_element_type=jnp.float32)
    acc = acc + jnp.dot(nc_ref[...], w1d_ref[...],
                        preferred_element_type=jnp.float32)
    acc = acc + b1_ref[...]
    h = jnp.maximum(acc, 0.0).astype(jnp.bfloat16)
    o_ref[...] = jnp.dot(h, w2_ref[...], preferred_element_type=jnp.float32)


def _mlp(x4, non_cat, w1, w1d, b1, w2):
    BM = 1024
    grid = (B // BM,)
    return pl.pallas_call(
        _mlp_body,
        grid=grid,
        in_specs=[
            pl.BlockSpec((BM // 8, 7, 8, 128), lambda i: (i, 0, 0, 0)),
            pl.BlockSpec((BM, D_DENSE), lambda i: (i, 0)),
            pl.BlockSpec((7, 128, H), lambda i: (0, 0, 0)),
            pl.BlockSpec((D_DENSE, H), lambda i: (0, 0)),
            pl.BlockSpec((1, H), lambda i: (0, 0)),
            pl.BlockSpec((H, 1), lambda i: (0, 0)),
        ],
        out_specs=pl.BlockSpec((BM, 1), lambda i: (i, 0)),
        out_shape=jax.ShapeDtypeStruct((B, 1), jnp.float32),
    )(x4, non_cat, w1, w1d, b1, w2)


def kernel(cat_input, non_cat_input, tables, W1, b1, W2, b2):
    tbl_flat = tables.reshape(NF * V, E)
    flat = cat_input + (jnp.arange(NF, dtype=jnp.int32) * V)[None, :]
    fr = flat.reshape(RB, 8, NF)
    main = (fr[:, :, :24].reshape(RB, 8, 6, 4)
            .transpose(0, 2, 1, 3).reshape(RB, 192))
    tail = jnp.pad(fr[:, :, 24:], ((0, 0), (0, 0), (0, 2))).reshape(RB, 32)
    idx = jnp.concatenate([main, tail], axis=1).reshape(NW * IRW, 128)

    rows = _sc_gather(tbl_flat, idx)
    x4 = rows.reshape(RB, 7, 8, 128)

    w1p = jnp.pad(W1[:NF * E], ((0, 64), (0, 0))).astype(jnp.bfloat16)
    out = _mlp(x4, non_cat_input.astype(jnp.bfloat16),
               w1p.reshape(7, 128, H),
               W1[NF * E:].astype(jnp.bfloat16),
               b1.reshape(1, H), W2.astype(jnp.bfloat16))
    return out.reshape(-1) + b2[0]


# speedup vs baseline: 1.2940x; 1.2940x over previous
import functools

import jax
import jax.numpy as jnp
from jax import lax
from jax.experimental import pallas as pl
from jax.experimental.pallas import tpu as pltpu
from jax.experimental.pallas import tpu_sc as plsc

B = 16384
NF = 26
V = 100000
E = 32
D_DENSE = 13
H = 1028

NW = 32
RB = B // 8
RBW = RB // NW
KRB = 4
MSTEPS = RBW // KRB
RPB = 224
IRW = RBW * RPB // 128
CPS = KRB * RPB // 128


def _sc_gather_body(tbl_hbm, idx_hbm, out_hbm, imv, g0, g1, gsem, os0, os1):
    wid = lax.axis_index("s") * 2 + lax.axis_index("c")
    pltpu.sync_copy(idx_hbm.at[pl.ds(wid * IRW, IRW)], imv)

    slots = (g0, g1)
    osems = (os0, os1)
    pending = [None, None]
    for i in range(MSTEPS):
        slot = i % 2
        if pending[slot] is not None:
            pending[slot].wait()
        g = slots[slot]
        descs = []
        for j in range(CPS):
            descs.append(pltpu.async_copy(
                tbl_hbm.at[imv.at[i * CPS + j]],
                g.at[pl.ds(j * 128, 128)], gsem))
        for d in descs:
            d.wait()
        out_row = (wid * RBW + i * KRB) * RPB
        pending[slot] = pltpu.async_copy(
            g, out_hbm.at[pl.ds(out_row, KRB * RPB)], osems[slot])
    for p in pending:
        if p is not None:
            p.wait()


@functools.partial(
    pl.kernel,
    out_type=jax.ShapeDtypeStruct((RB * RPB, E), jnp.float32),
    mesh=plsc.VectorSubcoreMesh(core_axis_name="c", subcore_axis_name="s"),
    compiler_params=pltpu.CompilerParams(use_tc_tiling_on_sc=False),
    scratch_types=[
        pltpu.VMEM((IRW, 128), jnp.int32),
        pltpu.VMEM((KRB * RPB, E), jnp.float32),
        pltpu.VMEM((KRB * RPB, E), jnp.float32),
        pltpu.SemaphoreType.DMA,
        pltpu.SemaphoreType.DMA,
        pltpu.SemaphoreType.DMA,
    ],
)
def _sc_gather(tbl_hbm, idx_hbm, out_hbm, imv, g0, g1, gsem, os0, os1):
    _sc_gather_body(tbl_hbm, idx_hbm, out_hbm, imv, g0, g1, gsem, os0, os1)


_MLP_BM = 1024


def _mlp_body(x_hbm, nc_ref, w1_ref, w1d_ref, b1_ref, w2_ref, o_ref,
              xbuf, sems):
    bm8 = _MLP_BM // 8
    i = pl.program_id(0)
    n = pl.num_programs(0)
    slot = lax.rem(i, 2)
    nslot = lax.rem(i + 1, 2)

    @pl.when(i == 0)
    def _():
        pltpu.make_async_copy(x_hbm.at[pl.ds(0, bm8)], xbuf.at[0],
                              sems.at[0]).start()

    @pl.when(i + 1 < n)
    def _():
        pltpu.make_async_copy(x_hbm.at[pl.ds((i + 1) * bm8, bm8)],
                              xbuf.at[nslot], sems.at[nslot]).start()

    pltpu.make_async_copy(x_hbm.at[pl.ds(i * bm8, bm8)], xbuf.at[slot],
                          sems.at[slot]).wait()

    acc = jnp.zeros((_MLP_BM, H), jnp.float32)
    for c in range(7):
        xc = xbuf[slot, :, c].reshape(_MLP_BM, 128)
        if c == 6:
            lane = lax.broadcasted_iota(jnp.int32, xc.shape, 1)
            xc = jnp.where(lane < 64, xc, 0.0)
        acc = acc + jnp.dot(xc.astype(jnp.bfloat16), w1_ref[c],
                            preferred_element_type=jnp.float32)
    acc = acc + jnp.dot(nc_ref[...], w1d_ref[...],
                        preferred_element_type=jnp.float32)
    acc = acc + b1_ref[...]
    h = jnp.maximum(acc, 0.0).astype(jnp.bfloat16)
    o_ref[...] = jnp.dot(h, w2_ref[...], preferred_element_type=jnp.float32)


def _mlp(x4, non_cat, w1, w1d, b1, w2):
    BM = _MLP_BM
    grid = (B // BM,)
    return pl.pallas_call(
        _mlp_body,
        grid=grid,
        in_specs=[
            pl.BlockSpec(memory_space=pl.ANY),
            pl.BlockSpec((BM, D_DENSE), lambda i: (i, 0)),
            pl.BlockSpec((7, 128, H), lambda i: (0, 0, 0)),
            pl.BlockSpec((D_DENSE, H), lambda i: (0, 0)),
            pl.BlockSpec((1, H), lambda i: (0, 0)),
            pl.BlockSpec((H, 1), lambda i: (0, 0)),
        ],
        out_specs=pl.BlockSpec((BM, 1), lambda i: (i, 0)),
        out_shape=jax.ShapeDtypeStruct((B, 1), jnp.float32),
        scratch_shapes=[
            pltpu.VMEM((2, BM // 8, 7, 8, 128), jnp.float32),
            pltpu.SemaphoreType.DMA((2,)),
        ],
    )(x4, non_cat, w1, w1d, b1, w2)


def kernel(cat_input, non_cat_input, tables, W1, b1, W2, b2):
    tbl_flat = tables.reshape(NF * V, E)
    flat = cat_input + (jnp.arange(NF, dtype=jnp.int32) * V)[None, :]
    fr = flat.reshape(RB, 8, NF)
    main = (fr[:, :, :24].reshape(RB, 8, 6, 4)
            .transpose(0, 2, 1, 3).reshape(RB, 192))
    tail = jnp.concatenate([fr[:, :, 24:], fr[:, :, 24:]],
                           axis=2).reshape(RB, 32)
    idx = jnp.concatenate([main, tail], axis=1).reshape(NW * IRW, 128)

    x4 = _sc_gather(tbl_flat, idx).reshape(RB, 7, 8, 128)

    w1p = jnp.pad(W1[:NF * E], ((0, 64), (0, 0))).astype(jnp.bfloat16)
    out = _mlp(x4, non_cat_input.astype(jnp.bfloat16),
               w1p.reshape(7, 128, H),
               W1[NF * E:].astype(jnp.bfloat16),
               b1.reshape(1, H), W2.astype(jnp.bfloat16))
    return out.reshape(-1) + b2[0]


# speedup vs baseline: 1.2954x; 1.0011x over previous
import functools

import jax
import jax.numpy as jnp
from jax import lax
from jax.experimental import pallas as pl
from jax.experimental.pallas import tpu as pltpu
from jax.experimental.pallas import tpu_sc as plsc

B = 16384
NF = 26
V = 100000
E = 32
D_DENSE = 13
H = 1028

NW = 32
RB = B // 8
RBW = RB // NW
KRB = 4
MSTEPS = RBW // KRB
RPB = 224
IRW = RBW * RPB // 128
CPS = KRB * RPB // 128
GROWS = KRB * RPB
OROWS = GROWS // 4


def _repack(g32, g128):
    def body(r, _):
        for q in range(4):
            for h in range(2):
                v = g32[4 * r + q, pl.ds(16 * h, 16)]
                g128[r, pl.ds(32 * q + 16 * h, 16)] = v
        return 0

    lax.fori_loop(0, OROWS, body, 0)


def _sc_gather_body(tbl_hbm, idx_hbm, out_hbm, imv, g0, g1, g128,
                    gsem, osem):
    wid = lax.axis_index("s") * 2 + lax.axis_index("c")
    pltpu.sync_copy(idx_hbm.at[pl.ds(wid * IRW, IRW)], imv)

    slots = (g0, g1)

    def fire(i):
        g = slots[i % 2]
        descs = []
        for j in range(CPS):
            descs.append(pltpu.async_copy(
                tbl_hbm.at[imv.at[i * CPS + j]],
                g.at[pl.ds(j * 128, 128)], gsem))
        return descs

    pend_out = None
    descs = fire(0)
    for i in range(MSTEPS):
        for d in descs:
            d.wait()
        if i + 1 < MSTEPS:
            descs = fire(i + 1)
        if pend_out is not None:
            pend_out.wait()
        _repack(slots[i % 2], g128)
        out_row = (wid * RBW + i * KRB) * 56
        pend_out = pltpu.async_copy(
            g128, out_hbm.at[pl.ds(out_row, OROWS)], osem)
    pend_out.wait()


@functools.partial(
    pl.kernel,
    out_type=jax.ShapeDtypeStruct((RB * 56, 128), jnp.float32),
    mesh=plsc.VectorSubcoreMesh(core_axis_name="c", subcore_axis_name="s"),
    compiler_params=pltpu.CompilerParams(use_tc_tiling_on_sc=False),
    scratch_types=[
        pltpu.VMEM((IRW, 128), jnp.int32),
        pltpu.VMEM((GROWS, E), jnp.float32),
        pltpu.VMEM((GROWS, E), jnp.float32),
        pltpu.VMEM((OROWS, 128), jnp.float32),
        pltpu.SemaphoreType.DMA,
        pltpu.SemaphoreType.DMA,
    ],
)
def _sc_gather(tbl_hbm, idx_hbm, out_hbm, imv, g0, g1, g128, gsem, osem):
    _sc_gather_body(tbl_hbm, idx_hbm, out_hbm, imv, g0, g1, g128, gsem, osem)


_MLP_BM = 1024
_XR = _MLP_BM * 7 // 8


def _mlp_body(x_hbm, nc_ref, w1_ref, w1d_ref, b1_ref, w2_ref, o_ref,
              xbuf, sems):
    rows = _MLP_BM * 7
    i = pl.program_id(0)
    n = pl.num_programs(0)
    slot = lax.rem(i, 2)
    nslot = lax.rem(i + 1, 2)

    @pl.when(i == 0)
    def _():
        pltpu.make_async_copy(x_hbm.at[pl.ds(0, rows)], xbuf.at[0],
                              sems.at[0]).start()

    @pl.when(i + 1 < n)
    def _():
        pltpu.make_async_copy(x_hbm.at[pl.ds((i + 1) * rows, rows)],
                              xbuf.at[nslot], sems.at[nslot]).start()

    pltpu.make_async_copy(x_hbm.at[pl.ds(i * rows, rows)], xbuf.at[slot],
                          sems.at[slot]).wait()

    xall = xbuf[slot].reshape(_MLP_BM // 8, 7, 8, 128)
    acc = jnp.zeros((_MLP_BM, H), jnp.float32)
    for c in range(7):
        xc = xall[:, c].reshape(_MLP_BM, 128)
        if c == 6:
            lane = lax.broadcasted_iota(jnp.int32, xc.shape, 1)
            xc = jnp.where(lane < 64, xc, 0.0)
        acc = acc + jnp.dot(xc.astype(jnp.bfloat16), w1_ref[c],
                            preferred_element_type=jnp.float32)
    acc = acc + jnp.dot(nc_ref[...], w1d_ref[...],
                        preferred_element_type=jnp.float32)
    acc = acc + b1_ref[...]
    h = jnp.maximum(acc, 0.0).astype(jnp.bfloat16)
    o_ref[...] = jnp.dot(h, w2_ref[...], preferred_element_type=jnp.float32)


def _mlp(x2d, non_cat, w1, w1d, b1, w2):
    BM = _MLP_BM
    grid = (B // BM,)
    return pl.pallas_call(
        _mlp_body,
        grid=grid,
        in_specs=[
            pl.BlockSpec(memory_space=pl.ANY),
            pl.BlockSpec((BM, D_DENSE), lambda i: (i, 0)),
            pl.BlockSpec((7, 128, H), lambda i: (0, 0, 0)),
            pl.BlockSpec((D_DENSE, H), lambda i: (0, 0)),
            pl.BlockSpec((1, H), lambda i: (0, 0)),
            pl.BlockSpec((H, 1), lambda i: (0, 0)),
        ],
        out_specs=pl.BlockSpec((BM, 1), lambda i: (i, 0)),
        out_shape=jax.ShapeDtypeStruct((B, 1), jnp.float32),
        scratch_shapes=[
            pltpu.VMEM((2, BM * 7, 128), jnp.float32),
            pltpu.SemaphoreType.DMA((2,)),
        ],
    )(x2d, non_cat, w1, w1d, b1, w2)


def kernel(cat_input, non_cat_input, tables, W1, b1, W2, b2):
    tbl_flat = tables.reshape(NF * V, E)
    flat = cat_input + (jnp.arange(NF, dtype=jnp.int32) * V)[None, :]
    fr = flat.reshape(RB, 8, NF)
    main = (fr[:, :, :24].reshape(RB, 8, 6, 4)
            .transpose(0, 2, 1, 3).reshape(RB, 192))
    tail = jnp.concatenate([fr[:, :, 24:], fr[:, :, 24:]],
                           axis=2).reshape(RB, 32)
    idx = jnp.concatenate([main, tail], axis=1).reshape(NW * IRW, 128)

    x2d = _sc_gather(tbl_flat, idx)

    w1p = jnp.pad(W1[:NF * E], ((0, 64), (0, 0))).astype(jnp.bfloat16)
    out = _mlp(x2d, non_cat_input.astype(jnp.bfloat16),
               w1p.reshape(7, 128, H),
               W1[NF * E:].astype(jnp.bfloat16),
               b1.reshape(1, H), W2.astype(jnp.bfloat16))
    return out.reshape(-1) + b2[0]
